# baseline (speedup 1.0000x reference)
import functools

import jax
import jax.numpy as jnp
from jax import lax
from jax.experimental import pallas as pl
from jax.experimental.pallas import tpu as pltpu
from jax.experimental.pallas import tpu_sc as plsc

H = 4
E = 8
K = 2
D_IN = 1024
FEAT = 1024
N = 8192
HID = 32 * H
BN = 1024
NSEL = H * K

_NEG = -1e30
_L = 16


def _sc_routing(sp_hbm, out_hbm, sp_v, out_v):
    c = lax.axis_index("c")
    s = lax.axis_index("s")

    @pl.when(jnp.logical_and(c == 0, s == 0))
    def _():
        pltpu.sync_copy(sp_hbm, sp_v)
        lanes = lax.iota(jnp.int32, _L)
        i0s, i1s, deltas = [], [], []
        for h in range(H):
            row = sp_v[h, :]
            m0 = row[0]
            i0 = jnp.int32(0)
            for e in range(1, E):
                ve = row[e]
                take = ve > m0
                m0 = jnp.where(take, ve, m0)
                i0 = jnp.where(take, e, i0)
            m1 = jnp.float32(_NEG)
            i1 = jnp.int32(0)
            for e in range(E):
                ve = row[e]
                take = jnp.logical_and(ve > m1, e != i0)
                m1 = jnp.where(take, ve, m1)
                i1 = jnp.where(take, e, i1)
            i0s.append(i0)
            i1s.append(i1)
            deltas.append(m1 - m0)
        dvec = jnp.zeros((_L,), jnp.float32)
        for h in range(H):
            dvec = jnp.where(lanes == h, deltas[h], dvec)
        ev = jnp.exp(dvec)
        pv = ev / (1.0 + ev)
        p1s = [pv[h] for h in range(H)]
        p0s = [1.0 - p1s[h] for h in range(H)]
        idx_acc = jnp.zeros((_L,), jnp.float32)
        probs_acc = jnp.zeros((_L,), jnp.float32)
        for h in range(H):
            idx_acc = (idx_acc
                       + jnp.where(lanes == 2 * h, i0s[h].astype(jnp.float32),
                                   0.0)
                       + jnp.where(lanes == 2 * h + 1,
                                   i1s[h].astype(jnp.float32), 0.0))
            probs_acc = (probs_acc + jnp.where(lanes == 2 * h, p0s[h], 0.0)
                         + jnp.where(lanes == 2 * h + 1, p1s[h], 0.0))
        reg = jnp.float32(0.0)
        for a in range(H):
            gaa = p0s[a] * p0s[a] + p1s[a] * p1s[a]
            d = gaa - 1.0
            reg = reg + d * d
            for b2 in range(a + 1, H):
                gab = jnp.float32(0.0)
                for ia, pa in ((i0s[a], p0s[a]), (i1s[a], p1s[a])):
                    for ib, pb in ((i0s[b2], p0s[b2]), (i1s[b2], p1s[b2])):
                        gab = gab + jnp.where(ia == ib, pa * pb, 0.0)
                reg = reg + 2.0 * gab * gab
        out_v[0, :] = idx_acc
        out_v[1, :] = probs_acc
        out_v[2, :] = jnp.where(lanes == 0, reg, 0.0)
        pltpu.sync_copy(out_v, out_hbm)


def _fused_kernel(idx_ref, probs_ref, W_blk, W1_blk, b_blk, b1_blk, x_blk,
                  W2_blk, b2_blk, woutT_blk, bout_blk, out_ref, M_sc,
                  beff_sc):
    s = pl.program_id(0)

    @pl.when(s == 0)
    def _():
        M_sc[...] = jnp.zeros_like(M_sc)
        beff_sc[...] = b1_blk[...]

    @pl.when(s < NSEL)
    def _():
        p = probs_ref[s // K, s % K]
        W1m = W1_blk[0]
        M_sc[...] += p * jnp.dot(W_blk[0, 0], W1m,
                                 preferred_element_type=jnp.float32)
        beff_sc[...] += p * jnp.dot(b_blk[0], W1m,
                                    preferred_element_type=jnp.float32)

    @pl.when(s >= NSEL)
    def _():
        z1 = jnp.dot(x_blk[...], M_sc[...],
                     preferred_element_type=jnp.float32) + beff_sc[...]
        h1 = jax.nn.softplus(z1)
        z2 = jnp.dot(h1, W2_blk[...],
                     preferred_element_type=jnp.float32) + b2_blk[...]
        h2 = jax.nn.softplus(z2)
        out_ref[...] = (jnp.sum(h2 * woutT_blk[...], axis=1, keepdims=True)
                        + bout_blk[...])


def kernel(x, scaling_params, W, b, W1, b1, W2, b2, Wout, bout):
    f32 = jnp.float32

    sp_pad = jnp.pad(scaling_params, ((0, 0), (0, _L - E)),
                     constant_values=_NEG)

    routing = functools.partial(
        pl.kernel,
        out_type=jax.ShapeDtypeStruct((3, _L), f32),
        mesh=plsc.VectorSubcoreMesh(core_axis_name="c", subcore_axis_name="s"),
        scratch_types=[
            pltpu.VMEM((H, _L), f32),
            pltpu.VMEM((3, _L), f32),
        ],
    )(_sc_routing)
    packed = routing(sp_pad)
    idx = packed[0, :NSEL].astype(jnp.int32).reshape(H, K)
    probs = packed[1, :NSEL].reshape(H, K)
    reg = packed[2, 0]

    W1r = jnp.transpose(W1.reshape(FEAT, H, HID), (1, 0, 2))
    b_r = b.reshape(H * E, 1, FEAT)
    b1_r = b1.reshape(1, HID)

    def _w_map(s, idx_ref, pr_ref):
        sc = jnp.minimum(s, NSEL - 1)
        return sc // K, idx_ref[sc // K, sc % K], 0, 0

    def _b_map(s, idx_ref, pr_ref):
        sc = jnp.minimum(s, NSEL - 1)
        return (sc // K) * E + idx_ref[sc // K, sc % K], 0, 0

    grid_spec = pltpu.PrefetchScalarGridSpec(
        num_scalar_prefetch=2,
        grid=(NSEL + N // BN,),
        in_specs=[
            pl.BlockSpec((1, 1, D_IN, FEAT), _w_map),
            pl.BlockSpec((1, FEAT, HID),
                         lambda s, idx_ref, pr_ref: (
                             jnp.minimum(s // K, H - 1), 0, 0)),
            pl.BlockSpec((1, 1, FEAT), _b_map),
            pl.BlockSpec((1, HID), lambda s, idx_ref, pr_ref: (0, 0)),
            pl.BlockSpec((BN, D_IN),
                         lambda s, idx_ref, pr_ref: (
                             jnp.maximum(s - NSEL, 0), 0)),
            pl.BlockSpec((HID, HID), lambda s, idx_ref, pr_ref: (0, 0)),
            pl.BlockSpec((1, HID), lambda s, idx_ref, pr_ref: (0, 0)),
            pl.BlockSpec((1, HID), lambda s, idx_ref, pr_ref: (0, 0)),
            pl.BlockSpec((1, 1), lambda s, idx_ref, pr_ref: (0, 0)),
        ],
        out_specs=pl.BlockSpec((BN, 1),
                               lambda s, idx_ref, pr_ref: (
                                   jnp.maximum(s - NSEL, 0), 0)),
        scratch_shapes=[
            pltpu.VMEM((D_IN, HID), f32),
            pltpu.VMEM((1, HID), f32),
        ],
    )
    out = pl.pallas_call(
        _fused_kernel,
        grid_spec=grid_spec,
        out_shape=jax.ShapeDtypeStruct((N, 1), f32),
        compiler_params=pltpu.CompilerParams(
            dimension_semantics=("arbitrary",)),
    )(idx, probs, W, W1r, b_r, b1_r, x, W2, b2.reshape(1, HID),
      Wout.reshape(1, HID), bout.reshape(1, 1))

    return out, reg


# speedup vs baseline: 1.0964x; 1.0964x over previous
import functools

import jax
import jax.numpy as jnp
from jax import lax
from jax.experimental import pallas as pl
from jax.experimental.pallas import tpu as pltpu
from jax.experimental.pallas import tpu_sc as plsc

H = 4
E = 8
K = 2
D_IN = 1024
FEAT = 1024
N = 8192
HID = 32 * H
BN = 1024
NSEL = H * K

_NEG = -1e30
_L = 16


def _sc_routing(sp_hbm, out_hbm, sp_v, out_v):
    c = lax.axis_index("c")
    s = lax.axis_index("s")

    @pl.when(jnp.logical_and(c == 0, s == 0))
    def _():
        pltpu.sync_copy(sp_hbm, sp_v)
        lanes = lax.iota(jnp.int32, _L)
        half = [sp_v[pl.ds(0, _L)], sp_v[pl.ds(_L, _L)]]
        i0s, i1s, deltas = [], [], []
        for h in range(H):
            vec = half[h // 2]
            base = (h % 2) * E
            m0 = vec[base]
            i0 = jnp.int32(0)
            for e in range(1, E):
                ve = vec[base + e]
                take = ve > m0
                m0 = jnp.where(take, ve, m0)
                i0 = jnp.where(take, e, i0)
            m1 = jnp.float32(_NEG)
            i1 = jnp.int32(0)
            for e in range(E):
                ve = vec[base + e]
                take = jnp.logical_and(ve > m1, e != i0)
                m1 = jnp.where(take, ve, m1)
                i1 = jnp.where(take, e, i1)
            i0s.append(i0)
            i1s.append(i1)
            deltas.append(m1 - m0)
        dvec = jnp.zeros((_L,), jnp.float32)
        for h in range(H):
            dvec = jnp.where(lanes == h, deltas[h], dvec)
        ev = jnp.exp(dvec)
        pv = ev / (1.0 + ev)
        p1s = [pv[h] for h in range(H)]
        p0s = [1.0 - p1s[h] for h in range(H)]
        idx_acc = jnp.zeros((_L,), jnp.float32)
        probs_acc = jnp.zeros((_L,), jnp.float32)
        for h in range(H):
            idx_acc = (idx_acc
                       + jnp.where(lanes == 2 * h, i0s[h].astype(jnp.float32),
                                   0.0)
                       + jnp.where(lanes == 2 * h + 1,
                                   i1s[h].astype(jnp.float32), 0.0))
            probs_acc = (probs_acc + jnp.where(lanes == 2 * h, p0s[h], 0.0)
                         + jnp.where(lanes == 2 * h + 1, p1s[h], 0.0))
        reg = jnp.float32(0.0)
        for a in range(H):
            gaa = p0s[a] * p0s[a] + p1s[a] * p1s[a]
            d = gaa - 1.0
            reg = reg + d * d
            for b2 in range(a + 1, H):
                gab = jnp.float32(0.0)
                for ia, pa in ((i0s[a], p0s[a]), (i1s[a], p1s[a])):
                    for ib, pb in ((i0s[b2], p0s[b2]), (i1s[b2], p1s[b2])):
                        gab = gab + jnp.where(ia == ib, pa * pb, 0.0)
                reg = reg + 2.0 * gab * gab
        out_v[0, :] = idx_acc
        out_v[1, :] = probs_acc
        out_v[2, :] = jnp.where(lanes == 0, reg, 0.0)
        pltpu.sync_copy(out_v, out_hbm)


def _fused_kernel(pk_ref, W_blk, W1_blk, b_blk, b1_blk, x_blk,
                  W2_blk, b2_blk, woutT_blk, bout_blk, out_ref, reg_ref,
                  M_sc, beff_sc):
    s = pl.program_id(0)

    @pl.when(s == 0)
    def _():
        M_sc[...] = jnp.zeros_like(M_sc)
        beff_sc[...] = b1_blk[...]
        reg_ref[...] = jnp.zeros((1, 1), jnp.float32) + pk_ref[2, 0]

    @pl.when(s < NSEL)
    def _():
        p = pk_ref[1, jnp.minimum(s, NSEL - 1)]
        W1m = W1_blk[0]
        M_sc[...] += p * jnp.dot(W_blk[0, 0], W1m,
                                 preferred_element_type=jnp.float32)
        beff_sc[...] += p * jnp.dot(b_blk[0], W1m,
                                    preferred_element_type=jnp.float32)

    @pl.when(s >= NSEL)
    def _():
        z1 = jnp.dot(x_blk[...], M_sc[...],
                     preferred_element_type=jnp.float32) + beff_sc[...]
        h1 = jax.nn.softplus(z1)
        z2 = jnp.dot(h1, W2_blk[...],
                     preferred_element_type=jnp.float32) + b2_blk[...]
        h2 = jax.nn.softplus(z2)
        out_ref[...] = (jnp.sum(h2 * woutT_blk[...], axis=1, keepdims=True)
                        + bout_blk[...])


def kernel(x, scaling_params, W, b, W1, b1, W2, b2, Wout, bout):
    f32 = jnp.float32

    routing = functools.partial(
        pl.kernel,
        out_type=jax.ShapeDtypeStruct((3, _L), f32),
        mesh=plsc.VectorSubcoreMesh(core_axis_name="c", subcore_axis_name="s",
                                    num_cores=1),
        scratch_types=[
            pltpu.VMEM((H * E,), f32),
            pltpu.VMEM((3, _L), f32),
        ],
    )(_sc_routing)
    packed = routing(scaling_params.reshape(H * E))

    W1r = jnp.transpose(W1.reshape(FEAT, H, HID), (1, 0, 2))
    b_r = b.reshape(H * E, 1, FEAT)
    b1_r = b1.reshape(1, HID)

    def _w_map(s, pk_ref):
        sc = jnp.minimum(s, NSEL - 1)
        return sc // K, pk_ref[0, sc].astype(jnp.int32), 0, 0

    def _b_map(s, pk_ref):
        sc = jnp.minimum(s, NSEL - 1)
        return (sc // K) * E + pk_ref[0, sc].astype(jnp.int32), 0, 0

    grid_spec = pltpu.PrefetchScalarGridSpec(
        num_scalar_prefetch=1,
        grid=(NSEL + N // BN,),
        in_specs=[
            pl.BlockSpec((1, 1, D_IN, FEAT), _w_map),
            pl.BlockSpec((1, FEAT, HID),
                         lambda s, pk_ref: (jnp.minimum(s // K, H - 1), 0, 0)),
            pl.BlockSpec((1, 1, FEAT), _b_map),
            pl.BlockSpec((1, HID), lambda s, pk_ref: (0, 0)),
            pl.BlockSpec((BN, D_IN),
                         lambda s, pk_ref: (jnp.maximum(s - NSEL, 0), 0)),
            pl.BlockSpec((HID, HID), lambda s, pk_ref: (0, 0)),
            pl.BlockSpec((1, HID), lambda s, pk_ref: (0, 0)),
            pl.BlockSpec((1, HID), lambda s, pk_ref: (0, 0)),
            pl.BlockSpec((1, 1), lambda s, pk_ref: (0, 0)),
        ],
        out_specs=[
            pl.BlockSpec((BN, 1),
                         lambda s, pk_ref: (jnp.maximum(s - NSEL, 0), 0)),
            pl.BlockSpec((1, 1), lambda s, pk_ref: (0, 0)),
        ],
        scratch_shapes=[
            pltpu.VMEM((D_IN, HID), f32),
            pltpu.VMEM((1, HID), f32),
        ],
    )
    out, reg = pl.pallas_call(
        _fused_kernel,
        grid_spec=grid_spec,
        out_shape=(
            jax.ShapeDtypeStruct((N, 1), f32),
            jax.ShapeDtypeStruct((1, 1), f32),
        ),
        compiler_params=pltpu.CompilerParams(
            dimension_semantics=("arbitrary",)),
    )(packed, W, W1r, b_r, b1_r, x, W2, b2.reshape(1, HID),
      Wout.reshape(1, HID), bout.reshape(1, 1))

    return out, reg.reshape(())
